# baseline (speedup 1.0000x reference)
import jax
import jax.numpy as jnp
from jax.experimental import pallas as pl
from jax.experimental.pallas import tpu as pltpu

_T, _B, _N, _H = 25, 128, 207, 128


def _dot_t(a, b):
    return jax.lax.dot_general(a, b, (((1,), (1,)), ((), ())),
                               preferred_element_type=jnp.float32)


def _fused_kernel(xy, xf, xdt, xm, dts,
                  wy, wf, wdt, wm, whh, bih, bhh,
                  wt, bt, wd, bd,
                  f1f, f1dt, f1dec, f1b, f2, f2b,
                  out, gi_ref, dec_ref):
    H = _H
    gi_ref[:] = (_dot_t(xy[:], wy[:]) + _dot_t(xf[:], wf[:])
                 + _dot_t(xdt[:], wdt[:]) + _dot_t(xm[:], wm[:]) + bih[:])

    def step(ti, carry):
        h, target, decay_w = carry
        dtb = dts[pl.ds(ti * _B, _B), :]
        decayed = target + (h - target) * jnp.exp(-decay_w * dtb)
        gi = gi_ref[pl.ds(ti * _B, _B), :]
        gh = _dot_t(decayed, whh[:]) + bhh[:]
        r = jax.nn.sigmoid(gi[:, :H] + gh[:, :H])
        z = jax.nn.sigmoid(gi[:, H:2 * H] + gh[:, H:2 * H])
        n = jnp.tanh(gi[:, 2 * H:] + r * gh[:, 2 * H:])
        h_new = (1.0 - z) * n + z * decayed
        dec_ref[pl.ds(ti * _B, _B), :] = decayed
        target_new = _dot_t(h_new, wt[:]) + bt[:]
        decay_w_new = jax.nn.softplus(_dot_t(h_new, wd[:]) + bd[:])
        return h_new, target_new, decay_w_new

    zeros = jnp.zeros((_B, H), jnp.float32)
    carry = (zeros, zeros, zeros)
    for ti in range(_T):
        carry = step(ti, carry)

    h1 = jnp.maximum(_dot_t(xf[:], f1f[:]) + _dot_t(xdt[:], f1dt[:])
                     + _dot_t(dec_ref[:], f1dec[:]) + f1b[:], 0.0)
    out[:] = _dot_t(h1, f2[:]) + f2b[:]


def kernel(y, mask, features, delta_t, t, edge_index, edge_attr, num_graphs,
           W_ih, W_hh, b_ih, b_hh, W_target, b_target, W_decayw, b_decayw,
           fc1_W, fc1_b, fc2_W, fc2_b):
    T, B, N, H = _T, _B, _N, _H
    c1 = jnp.concatenate([y[:, :, 0], features[:, :, 0]], axis=1)
    c1 = c1.T.reshape(2 * T * B, N)
    c2 = jnp.concatenate([delta_t, mask], axis=1)
    c2 = c2.T.reshape(2 * T * B, N)
    xy, xf = c1[:T * B], c1[T * B:]
    xdt, xm = c2[:T * B], c2[T * B:]
    dts = jnp.concatenate([t[:, :1], t[:, 1:] - t[:, :-1]], axis=1)
    dts = dts.T.reshape(T * B, 1)

    pred = pl.pallas_call(
        _fused_kernel,
        out_shape=jax.ShapeDtypeStruct((T * B, N), jnp.float32),
        scratch_shapes=[
            pltpu.VMEM((T * B, 3 * H), jnp.float32),
            pltpu.VMEM((T * B, H), jnp.float32),
        ],
    )(xy, xf, xdt, xm, dts,
      W_ih[:, :N], W_ih[:, N:2 * N], W_ih[:, 2 * N:3 * N], W_ih[:, 3 * N:],
      W_hh, b_ih.reshape(1, -1), b_hh.reshape(1, -1),
      W_target, b_target.reshape(1, -1), W_decayw, b_decayw.reshape(1, -1),
      fc1_W[:, :N], fc1_W[:, N:2 * N], fc1_W[:, 2 * N:],
      fc1_b.reshape(1, -1), fc2_W, fc2_b.reshape(1, -1))

    return pred.reshape(T, B * N, 1)


# speedup vs baseline: 1.2928x; 1.2928x over previous
import jax
import jax.numpy as jnp
from jax.experimental import pallas as pl
from jax.experimental.pallas import tpu as pltpu

_T, _B, _N, _H = 25, 128, 207, 128


def _dot_t(a, b):
    return jax.lax.dot_general(a, b, (((1,), (1,)), ((), ())),
                               preferred_element_type=jnp.float32)


def _fused_kernel(xy, xf, xdt, xm, dts,
                  wy, wf, wdt, wm, whh, bih, bhh,
                  wt, bt, wd, bd,
                  f1f, f1dt, f1dec, f1b, f2, f2b,
                  out, gi_ref, dec_ref):
    H = _H
    gi_ref[:] = (_dot_t(xy[:], wy[:]) + _dot_t(xf[:], wf[:])
                 + _dot_t(xdt[:], wdt[:]) + _dot_t(xm[:], wm[:]) + bih[:])

    def step(ti, carry):
        h, target, decay_w = carry
        dtb = dts[pl.ds(ti * _B, _B), :]
        decayed = target + (h - target) * jnp.exp(-decay_w * dtb)
        gi = gi_ref[pl.ds(ti * _B, _B), :]
        gh = _dot_t(decayed, whh[:]) + bhh[:]
        r = jax.nn.sigmoid(gi[:, :H] + gh[:, :H])
        z = jax.nn.sigmoid(gi[:, H:2 * H] + gh[:, H:2 * H])
        n = jnp.tanh(gi[:, 2 * H:] + r * gh[:, 2 * H:])
        h_new = (1.0 - z) * n + z * decayed
        dec_ref[pl.ds(ti * _B, _B), :] = decayed
        target_new = _dot_t(h_new, wt[:]) + bt[:]
        decay_w_new = jax.nn.softplus(_dot_t(h_new, wd[:]) + bd[:])
        return h_new, target_new, decay_w_new

    zeros = jnp.zeros((_B, H), jnp.float32)
    carry = (zeros, zeros, zeros)
    for ti in range(_T):
        carry = step(ti, carry)

    h1 = jnp.maximum(_dot_t(xf[:], f1f[:]) + _dot_t(xdt[:], f1dt[:])
                     + _dot_t(dec_ref[:], f1dec[:]) + f1b[:], 0.0)
    out[:] = _dot_t(h1, f2[:]) + f2b[:]


def kernel(y, mask, features, delta_t, t, edge_index, edge_attr, num_graphs,
           W_ih, W_hh, b_ih, b_hh, W_target, b_target, W_decayw, b_decayw,
           fc1_W, fc1_b, fc2_W, fc2_b):
    T, B, N, H = _T, _B, _N, _H
    xy = y[:, :, 0].T.reshape(T * B, N)
    xf = features[:, :, 0].T.reshape(T * B, N)
    xdt = delta_t.T.reshape(T * B, N)
    xm = mask.T.reshape(T * B, N)
    dts = jnp.concatenate([t[:, :1], t[:, 1:] - t[:, :-1]], axis=1)
    dts = dts.T.reshape(T * B, 1)

    pred = pl.pallas_call(
        _fused_kernel,
        out_shape=jax.ShapeDtypeStruct((T * B, N), jnp.float32),
        scratch_shapes=[
            pltpu.VMEM((T * B, 3 * H), jnp.float32),
            pltpu.VMEM((T * B, H), jnp.float32),
        ],
    )(xy, xf, xdt, xm, dts,
      W_ih[:, :N], W_ih[:, N:2 * N], W_ih[:, 2 * N:3 * N], W_ih[:, 3 * N:],
      W_hh, b_ih.reshape(1, -1), b_hh.reshape(1, -1),
      W_target, b_target.reshape(1, -1), W_decayw, b_decayw.reshape(1, -1),
      fc1_W[:, :N], fc1_W[:, N:2 * N], fc1_W[:, 2 * N:],
      fc1_b.reshape(1, -1), fc2_W, fc2_b.reshape(1, -1))

    return pred.reshape(T, B * N, 1)
